# baseline (speedup 1.0000x reference)
import functools

import jax
import jax.numpy as jnp
from jax import lax
from jax.experimental import pallas as pl
from jax.experimental.pallas import tpu as pltpu
from jax.experimental.pallas import tpu_sc as plsc

_NC = 2
_NS = 16
_L = 16
_H = 64
_CHUNK = 2 * _H
_SBLEN = 1024
_RS = 512


@functools.lru_cache(maxsize=None)
def _build(nnz_pad, n_hd, d, per_tile):
    half = n_hd // _NC
    half_shift = half.bit_length() - 1
    rows_per_tile = half // _NS
    d_vregs = d // _L
    nsb = per_tile // _SBLEN

    def body(cols_hbm, rows_hbm, vals_hbm, m2_hbm, out_hbm,
             cstage, rstage, vstage, colr, rowr, valr,
             colloc, rowloc, valloc, gbuf, acc,
             sem_g, sem_s, sem_st):
        sc = lax.axis_index("c")
        tid = lax.axis_index("s")
        sc_vec = jnp.full((_L,), sc, jnp.int32)
        zero_f = jnp.zeros((_L,), jnp.float32)
        zero_i = jnp.zeros((_L,), jnp.int32)

        def zero_row(i, _):
            for j in range(d_vregs):
                gbuf[i, pl.ds(j * _L, _L)] = zero_f
            return 0
        lax.fori_loop(0, _CHUNK, zero_row, 0)
        for k in range(rows_per_tile // _CHUNK):
            pltpu.sync_copy(
                gbuf, acc.at[pl.ds(tid * rows_per_tile + k * _CHUNK, _CHUNK)])
        plsc.subcore_barrier()

        for h in range(2):
            for g in range(_H // _L):
                sl = pl.ds(h * _H + g * _L, _L)
                colloc[sl] = zero_i
                rowloc[h][pl.ds(g * _L, _L)] = zero_i
                valloc[sl] = zero_f
            pltpu.async_copy(gbuf.at[pl.ds(h * _H, _H)],
                             acc.at[rowloc[h]], add=True, sem=sem_s[h])

        def drain_scatter(h):
            pltpu.make_async_copy(gbuf.at[pl.ds(h * _H, _H)],
                                  acc.at[rowloc[h]], sem_s[h]).wait()

        def fire_gather(h, gath):
            drain_scatter(h)
            base = lax.bitwise_and(gath, _RS - 1)
            for g in range(_H // _L):
                sl = pl.ds(h * _H + g * _L, _L)
                src = pl.ds(base + g * _L, _L)
                colloc[sl] = colr[src]
                rowloc[h][pl.ds(g * _L, _L)] = rowr[src]
                valloc[sl] = valr[src]
            pltpu.async_copy(m2_hbm.at[colloc.at[pl.ds(h * _H, _H)]],
                             gbuf.at[pl.ds(h * _H, _H)], sem_g[h])

        def process_half(h):
            pltpu.make_async_copy(m2_hbm.at[colloc.at[pl.ds(h * _H, _H)]],
                                  gbuf.at[pl.ds(h * _H, _H)], sem_g[h]).wait()

            def scale(g, _):
                v16 = valloc[pl.ds(h * _H + g * _L, _L)]
                for i in range(_L):
                    v = v16[i]
                    row = h * _H + g * _L + i
                    for j in range(d_vregs):
                        s = pl.ds(j * _L, _L)
                        gbuf[row, s] = gbuf[row, s] * v
                return 0
            lax.fori_loop(0, _H // _L, scale, 0)
            pltpu.async_copy(gbuf.at[pl.ds(h * _H, _H)],
                             acc.at[rowloc[h]], add=True, sem=sem_s[h])

        def issue_stage(par, sb):
            base = tid * per_tile + sb * _SBLEN
            off = par * _SBLEN
            for hbm, st in ((cols_hbm, cstage), (rows_hbm, rstage),
                            (vals_hbm, vstage)):
                pltpu.async_copy(hbm.at[pl.ds(base, _SBLEN)],
                                 st.at[pl.ds(off, _SBLEN)], sem_st[par])

        def wait_stage(par, sb):
            base = tid * per_tile + sb * _SBLEN
            off = par * _SBLEN
            for hbm, st in ((cols_hbm, cstage), (rows_hbm, rstage),
                            (vals_hbm, vstage)):
                pltpu.make_async_copy(hbm.at[pl.ds(base, _SBLEN)],
                                      st.at[pl.ds(off, _SBLEN)],
                                      sem_st[par]).wait()

        def make_grp(off):
            def grp(g, carry):
                cnt, gath, done = carry
                sl = pl.ds(off + g * _L, _L)
                r = rstage[sl]
                mine = lax.shift_right_logical(r, half_shift) == sc_vec
                incl = plsc.cumsum(mine.astype(jnp.int32))
                pos = lax.bitwise_and(
                    incl + jnp.full((_L,), cnt - 1, jnp.int32), _RS - 1)
                plsc.store_scatter(colr, [pos], cstage[sl], mask=mine)
                plsc.store_scatter(rowr, [pos],
                                   lax.bitwise_and(r, half - 1), mask=mine)
                plsc.store_scatter(valr, [pos], vstage[sl], mask=mine)
                cnt = cnt + incl[_L - 1]

                fire = (cnt - gath) >= _H
                even = lax.bitwise_and(gath, _H) == 0

                @pl.when(jnp.logical_and(fire, even))
                def _():
                    fire_gather(0, gath)

                @pl.when(jnp.logical_and(fire, jnp.logical_not(even)))
                def _():
                    fire_gather(1, gath)

                gath = gath + jnp.where(fire, _H, 0).astype(jnp.int32)
                proc = (gath - done) >= _CHUNK

                @pl.when(proc)
                def _():
                    process_half(0)
                    process_half(1)

                done = done + jnp.where(proc, _CHUNK, 0).astype(jnp.int32)
                return cnt, gath, done

            return grp

        issue_stage(0, 0)

        def superblock(sb, carry):
            even = lax.bitwise_and(sb, 1) == 0
            more = sb + 1 < nsb

            @pl.when(even)
            def _():
                wait_stage(0, sb)

            @pl.when(jnp.logical_not(even))
            def _():
                wait_stage(1, sb)

            @pl.when(jnp.logical_and(even, more))
            def _():
                issue_stage(1, sb + 1)

            @pl.when(jnp.logical_and(jnp.logical_not(even), more))
            def _():
                issue_stage(0, sb + 1)

            off = lax.bitwise_and(sb, 1) * _SBLEN
            return lax.fori_loop(0, _SBLEN // _L, make_grp(off), carry)

        cnt, gath, done = lax.fori_loop(
            0, nsb, superblock,
            (jnp.int32(0), jnp.int32(0), jnp.int32(0)))

        for k in range(_CHUNK // _L):
            tail = pl.ds(lax.bitwise_and(cnt + k * _L, _RS - 1), _L)
            colr[tail] = zero_i
            rowr[tail] = zero_i
            valr[tail] = zero_f

        owe = cnt > gath
        even = lax.bitwise_and(gath, _H) == 0

        @pl.when(jnp.logical_and(owe, even))
        def _():
            fire_gather(0, gath)

        @pl.when(jnp.logical_and(owe, jnp.logical_not(even)))
        def _():
            fire_gather(1, gath)

        gath = gath + jnp.where(owe, _H, 0).astype(jnp.int32)

        @pl.when(gath - done >= _CHUNK)
        def _():
            process_half(0)
            process_half(1)

        @pl.when(gath - done == _H)
        def _():
            process_half(0)

        drain_scatter(0)
        drain_scatter(1)

        plsc.subcore_barrier()
        for k in range(rows_per_tile // _CHUNK):
            off = tid * rows_per_tile + k * _CHUNK
            pltpu.sync_copy(acc.at[pl.ds(off, _CHUNK)],
                            out_hbm.at[pl.ds(sc * half + off, _CHUNK)])

    return pl.kernel(
        body,
        out_type=jax.ShapeDtypeStruct((n_hd, d), jnp.float32),
        mesh=plsc.VectorSubcoreMesh(core_axis_name="c", subcore_axis_name="s"),
        scratch_types=[
            pltpu.VMEM((2 * _SBLEN,), jnp.int32),
            pltpu.VMEM((2 * _SBLEN,), jnp.int32),
            pltpu.VMEM((2 * _SBLEN,), jnp.float32),
            pltpu.VMEM((_RS,), jnp.int32),
            pltpu.VMEM((_RS,), jnp.int32),
            pltpu.VMEM((_RS,), jnp.float32),
            pltpu.VMEM((_CHUNK,), jnp.int32),
            [pltpu.VMEM((_H,), jnp.int32)] * 2,
            pltpu.VMEM((_CHUNK,), jnp.float32),
            pltpu.VMEM((_CHUNK, d), jnp.float32),
            pltpu.VMEM_SHARED((n_hd // _NC, d), jnp.float32),
            [pltpu.SemaphoreType.DMA] * 2,
            [pltpu.SemaphoreType.DMA] * 2,
            [pltpu.SemaphoreType.DMA] * 2,
        ],
        compiler_params=pltpu.CompilerParams(use_tc_tiling_on_sc=False,
                                             needs_layout_passes=False),
    )


def kernel(vertices, rows, cols, vals):
    if vertices.ndim != 3:
        vertices = vertices[None, :, :]
    b, m, k = vertices.shape
    d = b * k
    n_hd = m
    m2 = jnp.transpose(vertices, (1, 0, 2)).reshape(m, d)

    nnz = rows.shape[0]
    per_tile = -(-nnz // (_NS * _SBLEN)) * _SBLEN
    nnz_pad = per_tile * _NS
    pad = nnz_pad - nnz
    rows_p = jnp.concatenate([rows.astype(jnp.int32),
                              jnp.zeros((pad,), jnp.int32)])
    cols_p = jnp.concatenate([cols.astype(jnp.int32),
                              jnp.zeros((pad,), jnp.int32)])
    vals_p = jnp.concatenate([vals, jnp.zeros((pad,), jnp.float32)])

    out = _build(nnz_pad, n_hd, d, per_tile)(cols_p, rows_p, vals_p, m2)
    return jnp.transpose(out.reshape(n_hd, b, k), (1, 0, 2)).astype(jnp.float32)


# speedup vs baseline: 2.1289x; 2.1289x over previous
import functools

import jax
import jax.numpy as jnp
from jax import lax
from jax.experimental import pallas as pl
from jax.experimental.pallas import tpu as pltpu
from jax.experimental.pallas import tpu_sc as plsc

_NC = 2
_NS = 16
_L = 16
_H = 64
_CHUNK = 2 * _H
_SBLEN = 1536
_RS = 512


@functools.lru_cache(maxsize=None)
def _build(nnz_pad, n_hd, d, per_tile):
    half = n_hd // _NC
    half_shift = half.bit_length() - 1
    rows_per_tile = half // _NS
    d_bregs = d // (2 * _L)
    nsb = per_tile // _SBLEN

    def body(cols_hbm, rows_hbm, vals_hbm, m2_hbm, out_hbm,
             cstage, rstage, vstage, colr, rowr, valr,
             colloc, rowloc, valloc, gbuf, acc,
             sem_g, sem_s):
        sc = lax.axis_index("c")
        tid = lax.axis_index("s")
        sc_vec = jnp.full((_L,), sc, jnp.int32)
        zero_f = jnp.zeros((_L,), jnp.float32)
        zero_i = jnp.zeros((_L,), jnp.int32)
        zero_b = jnp.zeros((2 * _L,), jnp.bfloat16)

        def zero_row(i, _):
            for j in range(d_bregs):
                gbuf[i, pl.ds(j * 2 * _L, 2 * _L)] = zero_b
            return 0
        lax.fori_loop(0, _CHUNK, zero_row, 0)
        for k in range(rows_per_tile // _CHUNK):
            pltpu.sync_copy(
                gbuf, acc.at[pl.ds(tid * rows_per_tile + k * _CHUNK, _CHUNK)])
        plsc.subcore_barrier()

        for h in range(2):
            for g in range(_H // _L):
                sl = pl.ds(h * _H + g * _L, _L)
                colloc[sl] = zero_i
                rowloc[h][pl.ds(g * _L, _L)] = zero_i
                valloc[sl] = zero_f
            pltpu.async_copy(gbuf.at[pl.ds(h * _H, _H)],
                             acc.at[rowloc[h]], add=True, sem=sem_s[h])

        def drain_scatter(h):
            pltpu.make_async_copy(gbuf.at[pl.ds(h * _H, _H)],
                                  acc.at[rowloc[h]], sem_s[h]).wait()

        def fire_gather(h, gath):
            drain_scatter(h)
            base = lax.bitwise_and(gath, _RS - 1)
            for g in range(_H // _L):
                sl = pl.ds(h * _H + g * _L, _L)
                src = pl.ds(base + g * _L, _L)
                colloc[sl] = colr[src]
                rowloc[h][pl.ds(g * _L, _L)] = rowr[src]
                valloc[sl] = valr[src]
            pltpu.async_copy(m2_hbm.at[colloc.at[pl.ds(h * _H, _H)]],
                             gbuf.at[pl.ds(h * _H, _H)], sem_g[h])

        def process_half(h):
            pltpu.make_async_copy(m2_hbm.at[colloc.at[pl.ds(h * _H, _H)]],
                                  gbuf.at[pl.ds(h * _H, _H)], sem_g[h]).wait()

            def scale(g, _):
                v16 = valloc[pl.ds(h * _H + g * _L, _L)]
                for i in range(_L):
                    vf = lax.broadcast_in_dim(v16[i], (_L,), ())
                    vv = plsc.pack(vf, vf, format=plsc.PackFormat.INTERLEAVED)
                    row = h * _H + g * _L + i
                    for j in range(d_bregs):
                        s = pl.ds(j * 2 * _L, 2 * _L)
                        gbuf[row, s] = gbuf[row, s] * vv
                return 0
            lax.fori_loop(0, _H // _L, scale, 0)
            pltpu.async_copy(gbuf.at[pl.ds(h * _H, _H)],
                             acc.at[rowloc[h]], add=True, sem=sem_s[h])

        def superblock(sb, carry):
            base = tid * per_tile + sb * _SBLEN
            pltpu.sync_copy(cols_hbm.at[pl.ds(base, _SBLEN)], cstage)
            pltpu.sync_copy(rows_hbm.at[pl.ds(base, _SBLEN)], rstage)
            pltpu.sync_copy(vals_hbm.at[pl.ds(base, _SBLEN)], vstage)

            def grp(g, carry):
                cnt, gath, done = carry
                sl = pl.ds(g * _L, _L)
                r = rstage[sl]
                mine = lax.shift_right_logical(r, half_shift) == sc_vec
                incl = plsc.cumsum(mine.astype(jnp.int32))
                pos = lax.bitwise_and(
                    incl + jnp.full((_L,), cnt - 1, jnp.int32), _RS - 1)
                plsc.store_scatter(colr, [pos], cstage[sl], mask=mine)
                plsc.store_scatter(rowr, [pos],
                                   lax.bitwise_and(r, half - 1), mask=mine)
                plsc.store_scatter(valr, [pos], vstage[sl], mask=mine)
                cnt = cnt + incl[_L - 1]

                fire = (cnt - gath) >= _H
                even = lax.bitwise_and(gath, _H) == 0

                @pl.when(jnp.logical_and(fire, even))
                def _():
                    fire_gather(0, gath)

                @pl.when(jnp.logical_and(fire, jnp.logical_not(even)))
                def _():
                    fire_gather(1, gath)

                gath = gath + jnp.where(fire, _H, 0).astype(jnp.int32)
                proc = (gath - done) >= _CHUNK

                @pl.when(proc)
                def _():
                    process_half(0)
                    process_half(1)

                done = done + jnp.where(proc, _CHUNK, 0).astype(jnp.int32)
                return cnt, gath, done

            return lax.fori_loop(0, _SBLEN // _L, grp, carry)

        cnt, gath, done = lax.fori_loop(
            0, nsb, superblock,
            (jnp.int32(0), jnp.int32(0), jnp.int32(0)))

        for k in range(_CHUNK // _L):
            tail = pl.ds(lax.bitwise_and(cnt + k * _L, _RS - 1), _L)
            colr[tail] = zero_i
            rowr[tail] = zero_i
            valr[tail] = zero_f

        owe = cnt > gath
        even = lax.bitwise_and(gath, _H) == 0

        @pl.when(jnp.logical_and(owe, even))
        def _():
            fire_gather(0, gath)

        @pl.when(jnp.logical_and(owe, jnp.logical_not(even)))
        def _():
            fire_gather(1, gath)

        gath = gath + jnp.where(owe, _H, 0).astype(jnp.int32)

        @pl.when(gath - done >= _CHUNK)
        def _():
            process_half(0)
            process_half(1)

        @pl.when(gath - done == _H)
        def _():
            process_half(0)

        drain_scatter(0)
        drain_scatter(1)

        plsc.subcore_barrier()
        for k in range(rows_per_tile // _CHUNK):
            off = tid * rows_per_tile + k * _CHUNK
            pltpu.sync_copy(acc.at[pl.ds(off, _CHUNK)],
                            out_hbm.at[pl.ds(sc * half + off, _CHUNK)])

    return pl.kernel(
        body,
        out_type=jax.ShapeDtypeStruct((n_hd, d), jnp.bfloat16),
        mesh=plsc.VectorSubcoreMesh(core_axis_name="c", subcore_axis_name="s"),
        scratch_types=[
            pltpu.VMEM((_SBLEN,), jnp.int32),
            pltpu.VMEM((_SBLEN,), jnp.int32),
            pltpu.VMEM((_SBLEN,), jnp.float32),
            pltpu.VMEM((_RS,), jnp.int32),
            pltpu.VMEM((_RS,), jnp.int32),
            pltpu.VMEM((_RS,), jnp.float32),
            pltpu.VMEM((_CHUNK,), jnp.int32),
            [pltpu.VMEM((_H,), jnp.int32)] * 2,
            pltpu.VMEM((_CHUNK,), jnp.float32),
            pltpu.VMEM((_CHUNK, d), jnp.bfloat16),
            pltpu.VMEM_SHARED((n_hd // _NC, d), jnp.bfloat16),
            [pltpu.SemaphoreType.DMA] * 2,
            [pltpu.SemaphoreType.DMA] * 2,
        ],
        compiler_params=pltpu.CompilerParams(use_tc_tiling_on_sc=False,
                                             needs_layout_passes=False),
    )


def kernel(vertices, rows, cols, vals):
    if vertices.ndim != 3:
        vertices = vertices[None, :, :]
    b, m, k = vertices.shape
    d = b * k
    n_hd = m
    m2 = jnp.transpose(vertices, (1, 0, 2)).reshape(m, d).astype(jnp.bfloat16)

    nnz = rows.shape[0]
    per_tile = -(-nnz // (_NS * _SBLEN)) * _SBLEN
    nnz_pad = per_tile * _NS
    pad = nnz_pad - nnz
    rows_p = jnp.concatenate([rows.astype(jnp.int32),
                              jnp.zeros((pad,), jnp.int32)])
    cols_p = jnp.concatenate([cols.astype(jnp.int32),
                              jnp.zeros((pad,), jnp.int32)])
    vals_p = jnp.concatenate([vals, jnp.zeros((pad,), jnp.float32)])

    out = _build(nnz_pad, n_hd, d, per_tile)(cols_p, rows_p, vals_p, m2)
    return jnp.transpose(out.astype(jnp.float32).reshape(n_hd, b, k),
                         (1, 0, 2))


# speedup vs baseline: 2.5211x; 1.1842x over previous
import functools

import jax
import jax.numpy as jnp
from jax import lax
from jax.experimental import pallas as pl
from jax.experimental.pallas import tpu as pltpu
from jax.experimental.pallas import tpu_sc as plsc

_NC = 2
_NS = 16
_L = 16
_H = 64
_CHUNK = 2 * _H
_SBLEN = 1536
_RS = 512


@functools.lru_cache(maxsize=None)
def _build(nnz_pad, n_hd, d, per_tile):
    half = n_hd // _NC
    half_shift = half.bit_length() - 1
    rows_per_tile = half // _NS
    d_bregs = d // (2 * _L)
    nsb = per_tile // _SBLEN

    def body(cols_hbm, rows_hbm, vals_hbm, m2_hbm, out_hbm,
             cstage, rstage, vstage, colr, rowr, valr,
             colloc, rowloc, valloc, gbuf, acc,
             sem_g, sem_s):
        sc = lax.axis_index("c")
        tid = lax.axis_index("s")
        sc_vec = jnp.full((_L,), sc, jnp.int32)
        zero_f = jnp.zeros((_L,), jnp.float32)
        zero_i = jnp.zeros((_L,), jnp.int32)
        zero_b = jnp.zeros((2 * _L,), jnp.bfloat16)

        def zero_row(i, _):
            for j in range(d_bregs):
                gbuf[i, pl.ds(j * 2 * _L, 2 * _L)] = zero_b
            return 0
        lax.fori_loop(0, _CHUNK, zero_row, 0)
        for k in range(rows_per_tile // _CHUNK):
            pltpu.sync_copy(
                gbuf, acc.at[pl.ds(tid * rows_per_tile + k * _CHUNK, _CHUNK)])
        plsc.subcore_barrier()

        for h in range(2):
            for g in range(_H // _L):
                sl = pl.ds(h * _H + g * _L, _L)
                colloc[sl] = zero_i
                rowloc[h][pl.ds(g * _L, _L)] = zero_i
                valloc[sl] = zero_f
            pltpu.async_copy(gbuf.at[pl.ds(h * _H, _H)],
                             acc.at[rowloc[h]], add=True, sem=sem_s[h])

        def drain_scatter(h):
            pltpu.make_async_copy(gbuf.at[pl.ds(h * _H, _H)],
                                  acc.at[rowloc[h]], sem_s[h]).wait()

        def fire_gather(h, gath):
            drain_scatter(h)
            base = lax.bitwise_and(gath, _RS - 1)
            for g in range(_H // _L):
                sl = pl.ds(h * _H + g * _L, _L)
                src = pl.ds(base + g * _L, _L)
                colloc[sl] = colr[src]
                rowloc[h][pl.ds(g * _L, _L)] = rowr[src]
                valloc[sl] = valr[src]
            pltpu.async_copy(m2_hbm.at[colloc.at[pl.ds(h * _H, _H)]],
                             gbuf.at[pl.ds(h * _H, _H)], sem_g[h])

        def process_half(h):
            pltpu.make_async_copy(m2_hbm.at[colloc.at[pl.ds(h * _H, _H)]],
                                  gbuf.at[pl.ds(h * _H, _H)], sem_g[h]).wait()

            def scale(g, _):
                v16 = valloc[pl.ds(h * _H + g * _L, _L)]
                for i in range(_L):
                    vf = lax.broadcast_in_dim(v16[i], (_L,), ())
                    vv = plsc.pack(vf, vf, format=plsc.PackFormat.INTERLEAVED)
                    row = h * _H + g * _L + i
                    for j in range(d_bregs):
                        s = pl.ds(j * 2 * _L, 2 * _L)
                        gbuf[row, s] = gbuf[row, s] * vv
                return 0
            lax.fori_loop(0, _H // _L, scale, 0)
            pltpu.async_copy(gbuf.at[pl.ds(h * _H, _H)],
                             acc.at[rowloc[h]], add=True, sem=sem_s[h])

        def superblock(sb, carry):
            base = tid * per_tile + sb * _SBLEN
            pltpu.sync_copy(cols_hbm.at[pl.ds(base, _SBLEN)], cstage)
            pltpu.sync_copy(rows_hbm.at[pl.ds(base, _SBLEN)], rstage)
            pltpu.sync_copy(vals_hbm.at[pl.ds(base, _SBLEN)], vstage)

            def grp(g, carry):
                cnt, gath, done = carry
                sl = pl.ds(g * _L, _L)
                r = rstage[sl]
                mine = lax.shift_right_logical(r, half_shift) == sc_vec
                incl = plsc.cumsum(mine.astype(jnp.int32))
                pos = lax.bitwise_and(
                    incl + jnp.full((_L,), cnt - 1, jnp.int32), _RS - 1)
                plsc.store_scatter(colr, [pos], cstage[sl], mask=mine)
                plsc.store_scatter(rowr, [pos],
                                   lax.bitwise_and(r, half - 1), mask=mine)
                plsc.store_scatter(valr, [pos], vstage[sl], mask=mine)
                cnt = cnt + incl[_L - 1]

                fire = (cnt - gath) >= _H
                even = lax.bitwise_and(gath, _H) == 0

                @pl.when(jnp.logical_and(fire, even))
                def _():
                    fire_gather(0, gath)

                @pl.when(jnp.logical_and(fire, jnp.logical_not(even)))
                def _():
                    fire_gather(1, gath)

                gath = gath + jnp.where(fire, _H, 0).astype(jnp.int32)
                proc = (gath - done) >= _CHUNK

                @pl.when(proc)
                def _():
                    process_half(0)
                    process_half(1)

                done = done + jnp.where(proc, _CHUNK, 0).astype(jnp.int32)
                return cnt, gath, done

            return lax.fori_loop(0, _SBLEN // _L, grp, carry)

        cnt, gath, done = lax.fori_loop(
            0, nsb, superblock,
            (jnp.int32(0), jnp.int32(0), jnp.int32(0)))

        for k in range(_CHUNK // _L):
            tail = pl.ds(lax.bitwise_and(cnt + k * _L, _RS - 1), _L)
            colr[tail] = zero_i
            rowr[tail] = zero_i
            valr[tail] = zero_f

        owe = cnt > gath
        even = lax.bitwise_and(gath, _H) == 0

        @pl.when(jnp.logical_and(owe, even))
        def _():
            fire_gather(0, gath)

        @pl.when(jnp.logical_and(owe, jnp.logical_not(even)))
        def _():
            fire_gather(1, gath)

        gath = gath + jnp.where(owe, _H, 0).astype(jnp.int32)

        @pl.when(gath - done >= _CHUNK)
        def _():
            process_half(0)
            process_half(1)

        @pl.when(gath - done == _H)
        def _():
            process_half(0)

        drain_scatter(0)
        drain_scatter(1)

        plsc.subcore_barrier()
        for k in range(rows_per_tile // _CHUNK):
            off = tid * rows_per_tile + k * _CHUNK
            pltpu.sync_copy(acc.at[pl.ds(off, _CHUNK)],
                            out_hbm.at[pl.ds(sc * half + off, _CHUNK)])

    return pl.kernel(
        body,
        out_type=jax.ShapeDtypeStruct((n_hd, d), jnp.bfloat16),
        mesh=plsc.VectorSubcoreMesh(core_axis_name="c", subcore_axis_name="s"),
        scratch_types=[
            pltpu.VMEM((_SBLEN,), jnp.int32),
            pltpu.VMEM((_SBLEN,), jnp.int32),
            pltpu.VMEM((_SBLEN,), jnp.float32),
            pltpu.VMEM((_RS,), jnp.int32),
            pltpu.VMEM((_RS,), jnp.int32),
            pltpu.VMEM((_RS,), jnp.float32),
            pltpu.VMEM((_CHUNK,), jnp.int32),
            [pltpu.VMEM((_H,), jnp.int32)] * 2,
            pltpu.VMEM((_CHUNK,), jnp.float32),
            pltpu.VMEM((_CHUNK, d), jnp.bfloat16),
            pltpu.VMEM_SHARED((n_hd // _NC, d), jnp.bfloat16),
            [pltpu.SemaphoreType.DMA] * 2,
            [pltpu.SemaphoreType.DMA] * 2,
        ],
        compiler_params=pltpu.CompilerParams(use_tc_tiling_on_sc=False,
                                             needs_layout_passes=False),
    )


def kernel(vertices, rows, cols, vals):
    if vertices.ndim != 3:
        vertices = vertices[None, :, :]
    b, m, k = vertices.shape
    d = b * k
    n_hd = m
    m2 = jnp.transpose(vertices, (1, 0, 2)).reshape(m, d).astype(jnp.bfloat16)

    nnz = rows.shape[0]
    per_tile = -(-nnz // (_NS * _SBLEN)) * _SBLEN
    nnz_pad = per_tile * _NS
    pad = nnz_pad - nnz
    rows_p = jnp.concatenate([rows.astype(jnp.int32),
                              jnp.full((pad,), n_hd, jnp.int32)])
    cols_p = jnp.concatenate([cols.astype(jnp.int32),
                              jnp.zeros((pad,), jnp.int32)])
    vals_p = jnp.concatenate([vals, jnp.zeros((pad,), jnp.float32)])

    out = _build(nnz_pad, n_hd, d, per_tile)(cols_p, rows_p, vals_p, m2)
    return jnp.transpose(out.astype(jnp.float32).reshape(n_hd, b, k),
                         (1, 0, 2))


# speedup vs baseline: 2.6178x; 1.0384x over previous
import functools

import jax
import jax.numpy as jnp
from jax import lax
from jax.experimental import pallas as pl
from jax.experimental.pallas import tpu as pltpu
from jax.experimental.pallas import tpu_sc as plsc

_NC = 2
_NS = 16
_L = 16
_H = 64
_CHUNK = 2 * _H
_SBLEN = 1536
_RS = 512


@functools.lru_cache(maxsize=None)
def _build(nnz_pad, n_hd, d, per_tile):
    half = n_hd // _NC
    col_bits = (n_hd - 1).bit_length()
    loc_mask = half - 1
    col_mask = n_hd - 1
    sc_shift = col_bits + half.bit_length() - 1
    rows_per_tile = half // _NS
    d_bregs = d // (2 * _L)
    nsb = per_tile // _SBLEN
    npairs = nsb // 2

    def body(rc_hbm, vals_hbm, m2_hbm, out_hbm,
             rcstage, vstage, rcr, valr,
             colloc, rowloc, valloc, gbuf, acc,
             sem_g, sem_s, sem_st):
        sc = lax.axis_index("c")
        tid = lax.axis_index("s")
        sc_vec = jnp.full((_L,), sc, jnp.int32)
        zero_f = jnp.zeros((_L,), jnp.float32)
        zero_i = jnp.zeros((_L,), jnp.int32)
        zero_b = jnp.zeros((2 * _L,), jnp.bfloat16)
        colm = jnp.full((_L,), col_mask, jnp.int32)
        locm = jnp.full((_L,), loc_mask, jnp.int32)

        def zero_row(i, _):
            for j in range(d_bregs):
                gbuf[i, pl.ds(j * 2 * _L, 2 * _L)] = zero_b
            return 0
        lax.fori_loop(0, _CHUNK, zero_row, 0)
        for k in range(rows_per_tile // _CHUNK):
            pltpu.sync_copy(
                gbuf, acc.at[pl.ds(tid * rows_per_tile + k * _CHUNK, _CHUNK)])
        plsc.subcore_barrier()

        for h in range(2):
            for g in range(_H // _L):
                sl = pl.ds(h * _H + g * _L, _L)
                colloc[sl] = zero_i
                rowloc[h][pl.ds(g * _L, _L)] = zero_i
                valloc[sl] = zero_f
            pltpu.async_copy(gbuf.at[pl.ds(h * _H, _H)],
                             acc.at[rowloc[h]], add=True, sem=sem_s[h])

        def drain_scatter(h):
            pltpu.make_async_copy(gbuf.at[pl.ds(h * _H, _H)],
                                  acc.at[rowloc[h]], sem_s[h]).wait()

        def fire_gather(h, gath):
            drain_scatter(h)
            base = lax.bitwise_and(gath, _RS - 1)
            for g in range(_H // _L):
                sl = pl.ds(h * _H + g * _L, _L)
                src = pl.ds(base + g * _L, _L)
                rcv = rcr[src]
                colloc[sl] = lax.bitwise_and(rcv, colm)
                rowloc[h][pl.ds(g * _L, _L)] = lax.bitwise_and(
                    lax.shift_right_logical(rcv, col_bits), locm)
                valloc[sl] = valr[src]
            pltpu.async_copy(m2_hbm.at[colloc.at[pl.ds(h * _H, _H)]],
                             gbuf.at[pl.ds(h * _H, _H)], sem_g[h])

        def process_half(h):
            pltpu.make_async_copy(m2_hbm.at[colloc.at[pl.ds(h * _H, _H)]],
                                  gbuf.at[pl.ds(h * _H, _H)], sem_g[h]).wait()

            def scale(g, _):
                v16 = valloc[pl.ds(h * _H + g * _L, _L)]
                for i in range(_L):
                    vf = lax.broadcast_in_dim(v16[i], (_L,), ())
                    vv = plsc.pack(vf, vf, format=plsc.PackFormat.INTERLEAVED)
                    row = h * _H + g * _L + i
                    for j in range(d_bregs):
                        s = pl.ds(j * 2 * _L, 2 * _L)
                        gbuf[row, s] = gbuf[row, s] * vv
                return 0
            lax.fori_loop(0, _H // _L, scale, 0)
            pltpu.async_copy(gbuf.at[pl.ds(h * _H, _H)],
                             acc.at[rowloc[h]], add=True, sem=sem_s[h])

        def issue_stage(par, sb):
            base = tid * per_tile + sb * _SBLEN
            off = par * _SBLEN
            pltpu.async_copy(rc_hbm.at[pl.ds(base, _SBLEN)],
                             rcstage.at[pl.ds(off, _SBLEN)], sem_st[par])
            pltpu.async_copy(vals_hbm.at[pl.ds(base, _SBLEN)],
                             vstage.at[pl.ds(off, _SBLEN)], sem_st[par])

        def wait_stage(par, sb):
            base = tid * per_tile + sb * _SBLEN
            off = par * _SBLEN
            pltpu.make_async_copy(rc_hbm.at[pl.ds(base, _SBLEN)],
                                  rcstage.at[pl.ds(off, _SBLEN)],
                                  sem_st[par]).wait()
            pltpu.make_async_copy(vals_hbm.at[pl.ds(base, _SBLEN)],
                                  vstage.at[pl.ds(off, _SBLEN)],
                                  sem_st[par]).wait()

        def make_grp(off):
            def grp(g, carry):
                cnt, gath, done = carry
                sl = pl.ds(off + g * _L, _L)
                rc = rcstage[sl]
                mine = lax.shift_right_logical(rc, sc_shift) == sc_vec
                incl = plsc.cumsum(mine.astype(jnp.int32))
                pos = lax.bitwise_and(
                    incl + jnp.full((_L,), cnt - 1, jnp.int32), _RS - 1)
                plsc.store_scatter(rcr, [pos], rc, mask=mine)
                plsc.store_scatter(valr, [pos], vstage[sl], mask=mine)
                cnt = cnt + incl[_L - 1]

                fire = (cnt - gath) >= _H
                even = lax.bitwise_and(gath, _H) == 0

                @pl.when(jnp.logical_and(fire, even))
                def _():
                    fire_gather(0, gath)

                @pl.when(jnp.logical_and(fire, jnp.logical_not(even)))
                def _():
                    fire_gather(1, gath)

                gath = gath + jnp.where(fire, _H, 0).astype(jnp.int32)
                proc = (gath - done) >= _CHUNK

                @pl.when(proc)
                def _():
                    process_half(0)
                    process_half(1)

                done = done + jnp.where(proc, _CHUNK, 0).astype(jnp.int32)
                return cnt, gath, done

            return grp

        issue_stage(0, 0)

        def pair(p, carry):
            sb0 = 2 * p
            wait_stage(0, sb0)
            issue_stage(1, sb0 + 1)
            carry = lax.fori_loop(0, _SBLEN // _L, make_grp(0), carry)
            wait_stage(1, sb0 + 1)

            @pl.when(p + 1 < npairs)
            def _():
                issue_stage(0, sb0 + 2)

            return lax.fori_loop(0, _SBLEN // _L, make_grp(_SBLEN), carry)

        cnt, gath, done = lax.fori_loop(
            0, npairs, pair, (jnp.int32(0), jnp.int32(0), jnp.int32(0)))

        for k in range(_CHUNK // _L):
            tail = pl.ds(lax.bitwise_and(cnt + k * _L, _RS - 1), _L)
            rcr[tail] = zero_i
            valr[tail] = zero_f

        owe = cnt > gath
        even = lax.bitwise_and(gath, _H) == 0

        @pl.when(jnp.logical_and(owe, even))
        def _():
            fire_gather(0, gath)

        @pl.when(jnp.logical_and(owe, jnp.logical_not(even)))
        def _():
            fire_gather(1, gath)

        gath = gath + jnp.where(owe, _H, 0).astype(jnp.int32)

        @pl.when(gath - done >= _CHUNK)
        def _():
            process_half(0)
            process_half(1)

        @pl.when(gath - done == _H)
        def _():
            process_half(0)

        drain_scatter(0)
        drain_scatter(1)

        plsc.subcore_barrier()
        for k in range(rows_per_tile // _CHUNK):
            off = tid * rows_per_tile + k * _CHUNK
            pltpu.sync_copy(acc.at[pl.ds(off, _CHUNK)],
                            out_hbm.at[pl.ds(sc * half + off, _CHUNK)])

    return pl.kernel(
        body,
        out_type=jax.ShapeDtypeStruct((n_hd, d), jnp.bfloat16),
        mesh=plsc.VectorSubcoreMesh(core_axis_name="c", subcore_axis_name="s"),
        scratch_types=[
            pltpu.VMEM((2 * _SBLEN,), jnp.int32),
            pltpu.VMEM((2 * _SBLEN,), jnp.float32),
            pltpu.VMEM((_RS,), jnp.int32),
            pltpu.VMEM((_RS,), jnp.float32),
            pltpu.VMEM((_CHUNK,), jnp.int32),
            [pltpu.VMEM((_H,), jnp.int32)] * 2,
            pltpu.VMEM((_CHUNK,), jnp.float32),
            pltpu.VMEM((_CHUNK, d), jnp.bfloat16),
            pltpu.VMEM_SHARED((n_hd // _NC, d), jnp.bfloat16),
            [pltpu.SemaphoreType.DMA] * 2,
            [pltpu.SemaphoreType.DMA] * 2,
            [pltpu.SemaphoreType.DMA] * 2,
        ],
        compiler_params=pltpu.CompilerParams(use_tc_tiling_on_sc=False,
                                             needs_layout_passes=False),
    )


def kernel(vertices, rows, cols, vals):
    if vertices.ndim != 3:
        vertices = vertices[None, :, :]
    b, m, k = vertices.shape
    d = b * k
    n_hd = m
    col_bits = (n_hd - 1).bit_length()
    m2 = jnp.transpose(vertices, (1, 0, 2)).reshape(m, d).astype(jnp.bfloat16)

    nnz = rows.shape[0]
    per_tile = -(-nnz // (_NS * 2 * _SBLEN)) * 2 * _SBLEN
    nnz_pad = per_tile * _NS
    pad = nnz_pad - nnz
    rc = jnp.left_shift(rows.astype(jnp.int32), col_bits) | cols.astype(
        jnp.int32)
    rc_p = jnp.concatenate([rc, jnp.full((pad,), n_hd << col_bits,
                                         jnp.int32)])
    vals_p = jnp.concatenate([vals, jnp.zeros((pad,), jnp.float32)])

    out = _build(nnz_pad, n_hd, d, per_tile)(rc_p, vals_p, m2)
    return jnp.transpose(out.astype(jnp.float32).reshape(n_hd, b, k),
                         (1, 0, 2))


# speedup vs baseline: 2.6209x; 1.0012x over previous
import functools

import jax
import jax.numpy as jnp
from jax import lax
from jax.experimental import pallas as pl
from jax.experimental.pallas import tpu as pltpu
from jax.experimental.pallas import tpu_sc as plsc

_NC = 2
_NS = 16
_L = 16
_H = 64
_CHUNK = 2 * _H
_SBLEN = 1536
_RS = 512


@functools.lru_cache(maxsize=None)
def _build(nnz_pad, n_hd, d, per_tile):
    half = n_hd // _NC
    col_bits = (n_hd - 1).bit_length()
    loc_mask = half - 1
    col_mask = n_hd - 1
    sc_shift = col_bits + half.bit_length() - 1
    rows_per_tile = half // _NS
    d_bregs = d // (2 * _L)
    nsb = per_tile // _SBLEN
    npairs = nsb // 2

    def body(rc_hbm, vals_hbm, m2_hbm, out_hbm,
             rcstage, vstage, rcr, valr,
             colloc, rowloc, valloc, gbuf, acc,
             sem_g, sem_s, sem_st):
        sc = lax.axis_index("c")
        tid = lax.axis_index("s")
        sc_vec = jnp.full((_L,), sc, jnp.int32)
        zero_f = jnp.zeros((_L,), jnp.float32)
        zero_i = jnp.zeros((_L,), jnp.int32)
        zero_b = jnp.zeros((2 * _L,), jnp.bfloat16)
        colm = jnp.full((_L,), col_mask, jnp.int32)
        locm = jnp.full((_L,), loc_mask, jnp.int32)

        def zero_row(i, _):
            for j in range(d_bregs):
                gbuf[i, pl.ds(j * 2 * _L, 2 * _L)] = zero_b
            return 0
        lax.fori_loop(0, _CHUNK, zero_row, 0)
        for k in range(rows_per_tile // _CHUNK):
            pltpu.sync_copy(
                gbuf, acc.at[pl.ds(tid * rows_per_tile + k * _CHUNK, _CHUNK)])
        plsc.subcore_barrier()

        for h in range(2):
            for g in range(_H // _L):
                sl = pl.ds(h * _H + g * _L, _L)
                colloc[sl] = zero_i
                rowloc[h][pl.ds(g * _L, _L)] = zero_i
                valloc[sl] = zero_f
            pltpu.async_copy(gbuf.at[pl.ds(h * _H, _H)],
                             acc.at[rowloc[h]], add=True, sem=sem_s[h])

        def drain_scatter(h):
            pltpu.make_async_copy(gbuf.at[pl.ds(h * _H, _H)],
                                  acc.at[rowloc[h]], sem_s[h]).wait()

        def fire_gather(h, gath):
            drain_scatter(h)
            base = lax.bitwise_and(gath, _RS - 1)
            for g in range(_H // _L):
                sl = pl.ds(h * _H + g * _L, _L)
                src = pl.ds(base + g * _L, _L)
                rcv = rcr[src]
                colloc[sl] = lax.bitwise_and(rcv, colm)
                rowloc[h][pl.ds(g * _L, _L)] = lax.bitwise_and(
                    lax.shift_right_logical(rcv, col_bits), locm)
                valloc[sl] = valr[src]
            pltpu.async_copy(m2_hbm.at[colloc.at[pl.ds(h * _H, _H)]],
                             gbuf.at[pl.ds(h * _H, _H)], sem_g[h])

        def process_half(h):
            pltpu.make_async_copy(m2_hbm.at[colloc.at[pl.ds(h * _H, _H)]],
                                  gbuf.at[pl.ds(h * _H, _H)], sem_g[h]).wait()

            def scale(g, _):
                v16 = valloc[pl.ds(h * _H + g * _L, _L)]
                for i in range(_L):
                    vf = lax.broadcast_in_dim(v16[i], (_L,), ())
                    vv = plsc.pack(vf, vf, format=plsc.PackFormat.INTERLEAVED)
                    row = h * _H + g * _L + i
                    for j in range(d_bregs):
                        s = pl.ds(j * 2 * _L, 2 * _L)
                        gbuf[row, s] = gbuf[row, s] * vv
                return 0
            lax.fori_loop(0, _H // _L, scale, 0)
            pltpu.async_copy(gbuf.at[pl.ds(h * _H, _H)],
                             acc.at[rowloc[h]], add=True, sem=sem_s[h])

        def issue_stage(par, sb):
            base = tid * per_tile + sb * _SBLEN
            off = par * _SBLEN
            pltpu.async_copy(rc_hbm.at[pl.ds(base, _SBLEN)],
                             rcstage.at[pl.ds(off, _SBLEN)], sem_st[par])
            pltpu.async_copy(vals_hbm.at[pl.ds(base, _SBLEN)],
                             vstage.at[pl.ds(off, _SBLEN)], sem_st[par])

        def wait_stage(par, sb):
            base = tid * per_tile + sb * _SBLEN
            off = par * _SBLEN
            pltpu.make_async_copy(rc_hbm.at[pl.ds(base, _SBLEN)],
                                  rcstage.at[pl.ds(off, _SBLEN)],
                                  sem_st[par]).wait()
            pltpu.make_async_copy(vals_hbm.at[pl.ds(base, _SBLEN)],
                                  vstage.at[pl.ds(off, _SBLEN)],
                                  sem_st[par]).wait()

        def make_grp(off):
            def grp(g, carry):
                cnt, gath, done = carry
                sl = pl.ds(off + g * _L, _L)
                rc = rcstage[sl]
                mine = lax.shift_right_logical(rc, sc_shift) == sc_vec
                incl = plsc.cumsum(mine.astype(jnp.int32))
                pos = lax.bitwise_and(
                    incl + jnp.full((_L,), cnt - 1, jnp.int32), _RS - 1)
                plsc.store_scatter(rcr, [pos], rc, mask=mine)
                plsc.store_scatter(valr, [pos], vstage[sl], mask=mine)
                cnt = cnt + incl[_L - 1]

                fire = (cnt - gath) >= _H
                even = lax.bitwise_and(gath, _H) == 0

                @pl.when(jnp.logical_and(fire, even))
                def _():
                    fire_gather(0, gath)

                @pl.when(jnp.logical_and(fire, jnp.logical_not(even)))
                def _():
                    fire_gather(1, gath)

                gath = gath + jnp.where(fire, _H, 0).astype(jnp.int32)
                proc = (gath - done) >= _CHUNK

                @pl.when(proc)
                def _():
                    process_half(0)
                    process_half(1)

                done = done + jnp.where(proc, _CHUNK, 0).astype(jnp.int32)
                return cnt, gath, done

            return grp

        issue_stage(0, 0)

        def pair(p, carry):
            sb0 = 2 * p
            wait_stage(0, sb0)
            issue_stage(1, sb0 + 1)
            carry = lax.fori_loop(0, _SBLEN // _L, make_grp(0), carry)
            wait_stage(1, sb0 + 1)

            @pl.when(p + 1 < npairs)
            def _():
                issue_stage(0, sb0 + 2)

            return lax.fori_loop(0, _SBLEN // _L, make_grp(_SBLEN), carry)

        cnt, gath, done = lax.fori_loop(
            0, npairs, pair, (jnp.int32(0), jnp.int32(0), jnp.int32(0)))

        for k in range(_CHUNK // _L):
            tail = pl.ds(lax.bitwise_and(cnt + k * _L, _RS - 1), _L)
            rcr[tail] = zero_i
            valr[tail] = zero_f

        owe = cnt > gath
        even = lax.bitwise_and(gath, _H) == 0

        @pl.when(jnp.logical_and(owe, even))
        def _():
            fire_gather(0, gath)

        @pl.when(jnp.logical_and(owe, jnp.logical_not(even)))
        def _():
            fire_gather(1, gath)

        gath = gath + jnp.where(owe, _H, 0).astype(jnp.int32)

        @pl.when(gath - done >= _CHUNK)
        def _():
            process_half(0)
            process_half(1)

        @pl.when(gath - done == _H)
        def _():
            process_half(0)

        drain_scatter(0)
        drain_scatter(1)

        plsc.subcore_barrier()
        for k in range(rows_per_tile // _CHUNK):
            off = tid * rows_per_tile + k * _CHUNK
            pltpu.sync_copy(acc.at[pl.ds(off, _CHUNK)],
                            out_hbm.at[pl.ds(sc * half + off, _CHUNK)])

    return pl.kernel(
        body,
        out_type=jax.ShapeDtypeStruct((n_hd, d), jnp.bfloat16),
        mesh=plsc.VectorSubcoreMesh(core_axis_name="c", subcore_axis_name="s"),
        scratch_types=[
            pltpu.VMEM((2 * _SBLEN,), jnp.int32),
            pltpu.VMEM((2 * _SBLEN,), jnp.float32),
            pltpu.VMEM((_RS,), jnp.int32),
            pltpu.VMEM((_RS,), jnp.float32),
            pltpu.VMEM((_CHUNK,), jnp.int32),
            [pltpu.VMEM((_H,), jnp.int32)] * 2,
            pltpu.VMEM((_CHUNK,), jnp.float32),
            pltpu.VMEM((_CHUNK, d), jnp.bfloat16),
            pltpu.VMEM_SHARED((n_hd // _NC, d), jnp.bfloat16),
            [pltpu.SemaphoreType.DMA] * 2,
            [pltpu.SemaphoreType.DMA] * 2,
            [pltpu.SemaphoreType.DMA] * 2,
        ],
        compiler_params=pltpu.CompilerParams(use_tc_tiling_on_sc=False,
                                             needs_layout_passes=False),
    )


def kernel(vertices, rows, cols, vals):
    if vertices.ndim != 3:
        vertices = vertices[None, :, :]
    b, m, k = vertices.shape
    d = b * k
    n_hd = m
    col_bits = (n_hd - 1).bit_length()
    m2 = jnp.transpose(vertices.astype(jnp.bfloat16), (1, 0, 2)).reshape(m, d)

    nnz = rows.shape[0]
    per_tile = -(-nnz // (_NS * 2 * _SBLEN)) * 2 * _SBLEN
    nnz_pad = per_tile * _NS
    pad = nnz_pad - nnz
    rc = jnp.left_shift(rows.astype(jnp.int32), col_bits) | cols.astype(
        jnp.int32)
    rc_p = jnp.concatenate([rc, jnp.full((pad,), n_hd << col_bits,
                                         jnp.int32)])
    vals_p = jnp.concatenate([vals, jnp.zeros((pad,), jnp.float32)])

    out = _build(nnz_pad, n_hd, d, per_tile)(rc_p, vals_p, m2)
    return jnp.transpose(out.reshape(n_hd, b, k),
                         (1, 0, 2)).astype(jnp.float32)


# speedup vs baseline: 2.7762x; 1.0592x over previous
import functools

import jax
import jax.numpy as jnp
from jax import lax
from jax.experimental import pallas as pl
from jax.experimental.pallas import tpu as pltpu
from jax.experimental.pallas import tpu_sc as plsc

_NC = 2
_NS = 16
_L = 16
_H = 128
_CHUNK = 2 * _H
_SBLEN = 1536
_RS = 512


@functools.lru_cache(maxsize=None)
def _build(nnz_pad, n_hd, d, per_tile):
    half = n_hd // _NC
    col_bits = (n_hd - 1).bit_length()
    loc_mask = half - 1
    col_mask = n_hd - 1
    sc_shift = col_bits + half.bit_length() - 1
    rows_per_tile = half // _NS
    d_bregs = d // (2 * _L)
    nsb = per_tile // _SBLEN
    npairs = nsb // 2

    def body(rc_hbm, vals_hbm, m2_hbm, out_hbm,
             rcstage, vstage, rcr, valr,
             colloc, rowloc, valloc, gbuf, acc,
             sem_g, sem_s, sem_st):
        sc = lax.axis_index("c")
        tid = lax.axis_index("s")
        sc_vec = jnp.full((_L,), sc, jnp.int32)
        zero_f = jnp.zeros((_L,), jnp.float32)
        zero_i = jnp.zeros((_L,), jnp.int32)
        zero_b = jnp.zeros((2 * _L,), jnp.bfloat16)
        colm = jnp.full((_L,), col_mask, jnp.int32)
        locm = jnp.full((_L,), loc_mask, jnp.int32)

        def zero_row(i, _):
            for j in range(d_bregs):
                gbuf[i, pl.ds(j * 2 * _L, 2 * _L)] = zero_b
            return 0
        lax.fori_loop(0, _CHUNK, zero_row, 0)
        for k in range(rows_per_tile // _CHUNK):
            pltpu.sync_copy(
                gbuf, acc.at[pl.ds(tid * rows_per_tile + k * _CHUNK, _CHUNK)])
        plsc.subcore_barrier()

        for h in range(2):
            for g in range(_H // _L):
                sl = pl.ds(h * _H + g * _L, _L)
                colloc[sl] = zero_i
                rowloc[h][pl.ds(g * _L, _L)] = zero_i
                valloc[sl] = zero_f
            pltpu.async_copy(gbuf.at[pl.ds(h * _H, _H)],
                             acc.at[rowloc[h]], add=True, sem=sem_s[h])

        def drain_scatter(h):
            pltpu.make_async_copy(gbuf.at[pl.ds(h * _H, _H)],
                                  acc.at[rowloc[h]], sem_s[h]).wait()

        def fire_gather(h, gath):
            drain_scatter(h)
            base = lax.bitwise_and(gath, _RS - 1)
            for g in range(_H // _L):
                sl = pl.ds(h * _H + g * _L, _L)
                src = pl.ds(base + g * _L, _L)
                rcv = rcr[src]
                colloc[sl] = lax.bitwise_and(rcv, colm)
                rowloc[h][pl.ds(g * _L, _L)] = lax.bitwise_and(
                    lax.shift_right_logical(rcv, col_bits), locm)
                valloc[sl] = valr[src]
            pltpu.async_copy(m2_hbm.at[colloc.at[pl.ds(h * _H, _H)]],
                             gbuf.at[pl.ds(h * _H, _H)], sem_g[h])

        def process_half(h):
            pltpu.make_async_copy(m2_hbm.at[colloc.at[pl.ds(h * _H, _H)]],
                                  gbuf.at[pl.ds(h * _H, _H)], sem_g[h]).wait()

            def scale(g, _):
                v16 = valloc[pl.ds(h * _H + g * _L, _L)]
                for i in range(_L):
                    vf = lax.broadcast_in_dim(v16[i], (_L,), ())
                    vv = plsc.pack(vf, vf, format=plsc.PackFormat.INTERLEAVED)
                    row = h * _H + g * _L + i
                    for j in range(d_bregs):
                        s = pl.ds(j * 2 * _L, 2 * _L)
                        gbuf[row, s] = gbuf[row, s] * vv
                return 0
            lax.fori_loop(0, _H // _L, scale, 0)
            pltpu.async_copy(gbuf.at[pl.ds(h * _H, _H)],
                             acc.at[rowloc[h]], add=True, sem=sem_s[h])

        def issue_stage(par, sb):
            base = tid * per_tile + sb * _SBLEN
            off = par * _SBLEN
            pltpu.async_copy(rc_hbm.at[pl.ds(base, _SBLEN)],
                             rcstage.at[pl.ds(off, _SBLEN)], sem_st[par])
            pltpu.async_copy(vals_hbm.at[pl.ds(base, _SBLEN)],
                             vstage.at[pl.ds(off, _SBLEN)], sem_st[par])

        def wait_stage(par, sb):
            base = tid * per_tile + sb * _SBLEN
            off = par * _SBLEN
            pltpu.make_async_copy(rc_hbm.at[pl.ds(base, _SBLEN)],
                                  rcstage.at[pl.ds(off, _SBLEN)],
                                  sem_st[par]).wait()
            pltpu.make_async_copy(vals_hbm.at[pl.ds(base, _SBLEN)],
                                  vstage.at[pl.ds(off, _SBLEN)],
                                  sem_st[par]).wait()

        def make_grp(off):
            def grp(g, carry):
                cnt, gath, done = carry
                sl = pl.ds(off + g * _L, _L)
                rc = rcstage[sl]
                mine = lax.shift_right_logical(rc, sc_shift) == sc_vec
                incl = plsc.cumsum(mine.astype(jnp.int32))
                pos = lax.bitwise_and(
                    incl + jnp.full((_L,), cnt - 1, jnp.int32), _RS - 1)
                plsc.store_scatter(rcr, [pos], rc, mask=mine)
                plsc.store_scatter(valr, [pos], vstage[sl], mask=mine)
                cnt = cnt + incl[_L - 1]

                fire = (cnt - gath) >= _H
                even = lax.bitwise_and(gath, _H) == 0

                @pl.when(jnp.logical_and(fire, even))
                def _():
                    fire_gather(0, gath)

                @pl.when(jnp.logical_and(fire, jnp.logical_not(even)))
                def _():
                    fire_gather(1, gath)

                gath = gath + jnp.where(fire, _H, 0).astype(jnp.int32)
                proc = (gath - done) >= _CHUNK

                @pl.when(proc)
                def _():
                    process_half(0)
                    process_half(1)

                done = done + jnp.where(proc, _CHUNK, 0).astype(jnp.int32)
                return cnt, gath, done

            return grp

        issue_stage(0, 0)

        def pair(p, carry):
            sb0 = 2 * p
            wait_stage(0, sb0)
            issue_stage(1, sb0 + 1)
            carry = lax.fori_loop(0, _SBLEN // _L, make_grp(0), carry)
            wait_stage(1, sb0 + 1)

            @pl.when(p + 1 < npairs)
            def _():
                issue_stage(0, sb0 + 2)

            return lax.fori_loop(0, _SBLEN // _L, make_grp(_SBLEN), carry)

        cnt, gath, done = lax.fori_loop(
            0, npairs, pair, (jnp.int32(0), jnp.int32(0), jnp.int32(0)))

        for k in range(_CHUNK // _L):
            tail = pl.ds(lax.bitwise_and(cnt + k * _L, _RS - 1), _L)
            rcr[tail] = zero_i
            valr[tail] = zero_f

        owe = cnt > gath
        even = lax.bitwise_and(gath, _H) == 0

        @pl.when(jnp.logical_and(owe, even))
        def _():
            fire_gather(0, gath)

        @pl.when(jnp.logical_and(owe, jnp.logical_not(even)))
        def _():
            fire_gather(1, gath)

        gath = gath + jnp.where(owe, _H, 0).astype(jnp.int32)

        @pl.when(gath - done >= _CHUNK)
        def _():
            process_half(0)
            process_half(1)

        @pl.when(gath - done == _H)
        def _():
            process_half(0)

        drain_scatter(0)
        drain_scatter(1)

        plsc.subcore_barrier()
        for k in range(rows_per_tile // _CHUNK):
            off = tid * rows_per_tile + k * _CHUNK
            pltpu.sync_copy(acc.at[pl.ds(off, _CHUNK)],
                            out_hbm.at[pl.ds(sc * half + off, _CHUNK)])

    return pl.kernel(
        body,
        out_type=jax.ShapeDtypeStruct((n_hd, d), jnp.bfloat16),
        mesh=plsc.VectorSubcoreMesh(core_axis_name="c", subcore_axis_name="s"),
        scratch_types=[
            pltpu.VMEM((2 * _SBLEN,), jnp.int32),
            pltpu.VMEM((2 * _SBLEN,), jnp.float32),
            pltpu.VMEM((_RS,), jnp.int32),
            pltpu.VMEM((_RS,), jnp.float32),
            pltpu.VMEM((_CHUNK,), jnp.int32),
            [pltpu.VMEM((_H,), jnp.int32)] * 2,
            pltpu.VMEM((_CHUNK,), jnp.float32),
            pltpu.VMEM((_CHUNK, d), jnp.bfloat16),
            pltpu.VMEM_SHARED((n_hd // _NC, d), jnp.bfloat16),
            [pltpu.SemaphoreType.DMA] * 2,
            [pltpu.SemaphoreType.DMA] * 2,
            [pltpu.SemaphoreType.DMA] * 2,
        ],
        compiler_params=pltpu.CompilerParams(use_tc_tiling_on_sc=False,
                                             needs_layout_passes=False),
    )


def kernel(vertices, rows, cols, vals):
    if vertices.ndim != 3:
        vertices = vertices[None, :, :]
    b, m, k = vertices.shape
    d = b * k
    n_hd = m
    col_bits = (n_hd - 1).bit_length()
    m2 = jnp.transpose(vertices.astype(jnp.bfloat16), (1, 0, 2)).reshape(m, d)

    nnz = rows.shape[0]
    per_tile = -(-nnz // (_NS * 2 * _SBLEN)) * 2 * _SBLEN
    nnz_pad = per_tile * _NS
    pad = nnz_pad - nnz
    rc = jnp.left_shift(rows.astype(jnp.int32), col_bits) | cols.astype(
        jnp.int32)
    rc_p = jnp.concatenate([rc, jnp.full((pad,), n_hd << col_bits,
                                         jnp.int32)])
    vals_p = jnp.concatenate([vals, jnp.zeros((pad,), jnp.float32)])

    out = _build(nnz_pad, n_hd, d, per_tile)(rc_p, vals_p, m2)
    return jnp.transpose(out.reshape(n_hd, b, k),
                         (1, 0, 2)).astype(jnp.float32)
